# baseline (speedup 1.0000x reference)
import functools

import jax
import jax.numpy as jnp
from jax import lax
from jax.experimental import pallas as pl
from jax.experimental.pallas import tpu as pltpu
from jax.experimental.pallas import tpu_sc as plsc

_CHUNK = 80


@functools.partial(jax.jit, static_argnums=(3, 4, 5))
def _sc_scatter_add(e3, idx3, zeros, N, E, D):
    NC, NS = 2, 16
    DC = D // NC
    NCH = E // _CHUNK
    NB = NCH // NS
    assert NB * NS == NCH and (NB - 2) % 3 == 0
    RU = (N // 16 // NS) * 16
    REM = N - RU * NS
    mesh = plsc.VectorSubcoreMesh(core_axis_name="c", subcore_axis_name="s")

    @functools.partial(
        pl.kernel,
        mesh=mesh,
        out_type=jax.ShapeDtypeStruct((N, D), jnp.float32),
        scratch_types=[
            pltpu.VMEM((NB, 1, _CHUNK), jnp.int32),
            pltpu.VMEM((_CHUNK, DC), jnp.float32),
            pltpu.VMEM((_CHUNK, DC), jnp.float32),
            pltpu.VMEM((_CHUNK, DC), jnp.float32),
            pltpu.VMEM_SHARED((N, DC), jnp.float32),
            pltpu.SemaphoreType.DMA,
            pltpu.SemaphoreType.DMA,
            pltpu.SemaphoreType.DMA,
            pltpu.SemaphoreType.DMA,
            pltpu.SemaphoreType.DMA,
            pltpu.SemaphoreType.DMA,
        ],
    )
    def scatter(e_hbm, idx_hbm, z_hbm, out_hbm, idx_v,
                ebuf0, ebuf1, ebuf2, acc,
                gsem0, gsem1, gsem2, ssem0, ssem1, ssem2):
        ebuf = (ebuf0, ebuf1, ebuf2)
        gsem = (gsem0, gsem1, gsem2)
        ssem = (ssem0, ssem1, ssem2)
        c = lax.axis_index("c")
        s = lax.axis_index("s")
        col0 = c * DC
        base = s * NB
        rb = s * RU

        pltpu.sync_copy(z_hbm, ebuf0)
        for k in range(RU // _CHUNK):
            pltpu.sync_copy(ebuf0, acc.at[pl.ds(rb + k * _CHUNK, _CHUNK), :])
        r = RU % _CHUNK
        if r:
            pltpu.sync_copy(ebuf0.at[pl.ds(0, r), :],
                            acc.at[pl.ds(rb + RU - r, r), :])

        @pl.when(s == 0)
        def _():
            pltpu.sync_copy(ebuf0.at[pl.ds(0, REM), :],
                            acc.at[pl.ds(N - REM, REM), :])

        pltpu.sync_copy(idx_hbm.at[pl.ds(base, NB)], idx_v)

        def gather_start(j, b):
            pltpu.async_copy(
                e_hbm.at[base + j, :, pl.ds(col0, DC)], ebuf[b], gsem[b])

        def gather_wait(j, b):
            pltpu.make_async_copy(
                e_hbm.at[base + j, :, pl.ds(col0, DC)], ebuf[b], gsem[b]).wait()

        def scat_start(j, b):
            pltpu.async_copy(
                ebuf[b], acc.at[idx_v.at[j, 0]], ssem[b], add=True)

        def scat_wait(j, b):
            pltpu.make_async_copy(
                ebuf[b], acc.at[idx_v.at[j, 0]], ssem[b]).wait()

        gather_start(0, 0)
        gather_start(1, 1)

        plsc.subcore_barrier()

        gather_wait(0, 0)
        scat_start(0, 0)
        gather_start(2, 2)
        gather_wait(1, 1)
        scat_start(1, 1)

        def body(jj, carry):
            for u in range(3):
                j = 2 + jj * 3 + u
                b = (2 + u) % 3
                bn = u
                gather_wait(j, b)
                scat_start(j, b)
                scat_wait(j - 2, bn)
                gather_start((j + 1) % NB, bn)
            return carry

        lax.fori_loop(0, (NB - 2) // 3, body, 0)

        scat_wait(NB - 2, (NB - 2) % 3)
        scat_wait(NB - 1, (NB - 1) % 3)
        gather_wait(0, NB % 3)

        plsc.subcore_barrier()

        pltpu.sync_copy(acc.at[pl.ds(rb, RU), :],
                        out_hbm.at[pl.ds(rb, RU), pl.ds(col0, DC)])

        @pl.when(s == 0)
        def _():
            pltpu.sync_copy(acc.at[pl.ds(N - REM, REM), :],
                            out_hbm.at[pl.ds(N - REM, REM), pl.ds(col0, DC)])

    return scatter(e3, idx3, zeros)


def _mlp(acc, v, W1, b1, W2, b2):
    n, d = v.shape
    br = 1000
    dn = (((1,), (1,)), ((), ()))

    def body(acc_ref, v_ref, w1_ref, b1_ref, w2_ref, b2_ref, out_ref):
        x = acc_ref[...]
        h = lax.dot_general(x, w1_ref[...], dn, preferred_element_type=jnp.float32)
        h = h + b1_ref[...]
        h = jnp.where(h >= 0, h, h * 0.015)
        y = lax.dot_general(h, w2_ref[...], dn, preferred_element_type=jnp.float32)
        out_ref[...] = y + b2_ref[...] + v_ref[...]

    return pl.pallas_call(
        body,
        grid=(n // br,),
        in_specs=[
            pl.BlockSpec((br, d), lambda i: (i, 0)),
            pl.BlockSpec((br, d), lambda i: (i, 0)),
            pl.BlockSpec((d, d), lambda i: (0, 0)),
            pl.BlockSpec((1, d), lambda i: (0, 0)),
            pl.BlockSpec((d, d), lambda i: (0, 0)),
            pl.BlockSpec((1, d), lambda i: (0, 0)),
        ],
        out_specs=pl.BlockSpec((br, d), lambda i: (i, 0)),
        out_shape=jax.ShapeDtypeStruct((n, d), jnp.float32),
    )(acc, v, W1, b1.reshape(1, d), W2, b2.reshape(1, d))


def kernel(v, e, edge_index, W1, b1, W2, b2):
    n, d = v.shape
    ne = e.shape[0]
    e3 = e.reshape(ne // _CHUNK, _CHUNK, d)
    idx3 = edge_index[1].reshape(ne // _CHUNK, 1, _CHUNK)
    zeros = jnp.zeros((_CHUNK, d // 2), jnp.float32)
    acc = _sc_scatter_add(e3, idx3, zeros, n, ne, d)
    return _mlp(acc, v, W1, b1, W2, b2)


# speedup vs baseline: 1.1961x; 1.1961x over previous
import functools

import jax
import jax.numpy as jnp
from jax import lax
from jax.experimental import pallas as pl
from jax.experimental.pallas import tpu as pltpu
from jax.experimental.pallas import tpu_sc as plsc

_CHUNK = 128


@functools.partial(jax.jit, static_argnums=(3, 4, 5))
def _sc_scatter_add(e3, idx3, zeros, N, E, D):
    NC, NS = 2, 16
    DC = D // NC
    NCH = E // _CHUNK
    NB = NCH // NS
    XTRA = NCH - NB * NS
    assert (NB - 2) % 2 == 0
    RU = (N // 16 // NS) * 16
    REM = N - RU * NS
    mesh = plsc.VectorSubcoreMesh(core_axis_name="c", subcore_axis_name="s")

    @functools.partial(
        pl.kernel,
        mesh=mesh,
        out_type=jax.ShapeDtypeStruct((N, D), jnp.float32),
        scratch_types=[
            pltpu.VMEM((NB + 1, 1, _CHUNK), jnp.int32),
            pltpu.VMEM((_CHUNK, DC), jnp.float32),
            pltpu.VMEM((_CHUNK, DC), jnp.float32),
            pltpu.VMEM_SHARED((N, DC), jnp.float32),
            pltpu.SemaphoreType.DMA,
            pltpu.SemaphoreType.DMA,
            pltpu.SemaphoreType.DMA,
            pltpu.SemaphoreType.DMA,
        ],
    )
    def scatter(e_hbm, idx_hbm, z_hbm, out_hbm, idx_v,
                ebuf0, ebuf1, acc, gsem0, gsem1, ssem0, ssem1):
        ebuf = (ebuf0, ebuf1)
        gsem = (gsem0, gsem1)
        ssem = (ssem0, ssem1)
        c = lax.axis_index("c")
        s = lax.axis_index("s")
        col0 = c * DC
        base = s * NB + jnp.minimum(s, XTRA)
        rb = s * RU

        pltpu.sync_copy(z_hbm, ebuf0)
        for k in range(RU // _CHUNK):
            pltpu.sync_copy(ebuf0, acc.at[pl.ds(rb + k * _CHUNK, _CHUNK), :])
        r = RU % _CHUNK
        if r:
            pltpu.sync_copy(ebuf0.at[pl.ds(0, r), :],
                            acc.at[pl.ds(rb + RU - r, r), :])

        @pl.when(s == 0)
        def _():
            pltpu.sync_copy(ebuf0.at[pl.ds(0, REM), :],
                            acc.at[pl.ds(N - REM, REM), :])

        pltpu.sync_copy(idx_hbm.at[pl.ds(base, NB)], idx_v.at[pl.ds(0, NB)])

        @pl.when(s < XTRA)
        def _():
            pltpu.sync_copy(idx_hbm.at[pl.ds(base + NB, 1)],
                            idx_v.at[pl.ds(NB, 1)])

        def gather_start(j, b):
            pltpu.async_copy(
                e_hbm.at[base + j, :, pl.ds(col0, DC)], ebuf[b], gsem[b])

        def gather_wait(j, b):
            pltpu.make_async_copy(
                e_hbm.at[base + j, :, pl.ds(col0, DC)], ebuf[b], gsem[b]).wait()

        def scat_start(j, b):
            pltpu.async_copy(
                ebuf[b], acc.at[idx_v.at[j, 0]], ssem[b], add=True)

        def scat_wait(j, b):
            pltpu.make_async_copy(
                ebuf[b], acc.at[idx_v.at[j, 0]], ssem[b]).wait()

        gather_start(0, 0)
        gather_start(1, 1)

        plsc.subcore_barrier()

        gather_wait(0, 0)
        scat_start(0, 0)
        gather_wait(1, 1)
        scat_start(1, 1)
        scat_wait(0, 0)
        gather_start(2, 0)

        def body(jj, carry):
            for u in range(2):
                j = 2 + jj * 2 + u
                b = u
                bn = 1 - u
                gather_wait(j, b)
                scat_start(j, b)
                scat_wait(j - 1, bn)
                gather_start((j + 1) % NB, bn)
            return carry

        lax.fori_loop(0, (NB - 2) // 2, body, 0)

        scat_wait(NB - 1, 1)
        gather_wait(0, 0)

        @pl.when(s < XTRA)
        def _():
            pltpu.sync_copy(e_hbm.at[base + NB, :, pl.ds(col0, DC)], ebuf0)
            pltpu.sync_copy(ebuf0, acc.at[idx_v.at[NB, 0]], add=True)

        plsc.subcore_barrier()

        pltpu.sync_copy(acc.at[pl.ds(rb, RU), :],
                        out_hbm.at[pl.ds(rb, RU), pl.ds(col0, DC)])

        @pl.when(s == 0)
        def _():
            pltpu.sync_copy(acc.at[pl.ds(N - REM, REM), :],
                            out_hbm.at[pl.ds(N - REM, REM), pl.ds(col0, DC)])

    return scatter(e3, idx3, zeros)


def _mlp(acc, v, W1, b1, W2, b2):
    n, d = v.shape
    br = 1000
    dn = (((1,), (1,)), ((), ()))

    def body(acc_ref, v_ref, w1_ref, b1_ref, w2_ref, b2_ref, out_ref):
        x = acc_ref[...]
        h = lax.dot_general(x, w1_ref[...], dn, preferred_element_type=jnp.float32)
        h = h + b1_ref[...]
        h = jnp.where(h >= 0, h, h * 0.015)
        y = lax.dot_general(h, w2_ref[...], dn, preferred_element_type=jnp.float32)
        out_ref[...] = y + b2_ref[...] + v_ref[...]

    return pl.pallas_call(
        body,
        grid=(n // br,),
        in_specs=[
            pl.BlockSpec((br, d), lambda i: (i, 0)),
            pl.BlockSpec((br, d), lambda i: (i, 0)),
            pl.BlockSpec((d, d), lambda i: (0, 0)),
            pl.BlockSpec((1, d), lambda i: (0, 0)),
            pl.BlockSpec((d, d), lambda i: (0, 0)),
            pl.BlockSpec((1, d), lambda i: (0, 0)),
        ],
        out_specs=pl.BlockSpec((br, d), lambda i: (i, 0)),
        out_shape=jax.ShapeDtypeStruct((n, d), jnp.float32),
    )(acc, v, W1, b1.reshape(1, d), W2, b2.reshape(1, d))


def kernel(v, e, edge_index, W1, b1, W2, b2):
    n, d = v.shape
    ne = e.shape[0]
    e3 = e.reshape(ne // _CHUNK, _CHUNK, d)
    idx3 = edge_index[1].reshape(ne // _CHUNK, 1, _CHUNK)
    zeros = jnp.zeros((_CHUNK, d // 2), jnp.float32)
    acc = _sc_scatter_add(e3, idx3, zeros, n, ne, d)
    return _mlp(acc, v, W1, b1, W2, b2)


# speedup vs baseline: 1.2021x; 1.0050x over previous
import functools

import jax
import jax.numpy as jnp
from jax import lax
from jax.experimental import pallas as pl
from jax.experimental.pallas import tpu as pltpu
from jax.experimental.pallas import tpu_sc as plsc

_CHUNK = 128


@functools.partial(jax.jit, static_argnums=(3, 4, 5))
def _sc_scatter_add(e3, idx3, zeros, N, E, D):
    NC, NS = 2, 16
    DC = D // NC
    NCH = E // _CHUNK
    NB = NCH // NS
    XTRA = NCH - NB * NS
    RU = (N // 16 // NS) * 16
    REM = N - RU * NS
    mesh = plsc.VectorSubcoreMesh(core_axis_name="c", subcore_axis_name="s")

    @functools.partial(
        pl.kernel,
        mesh=mesh,
        out_type=jax.ShapeDtypeStruct((N, D), jnp.float32),
        scratch_types=[
            pltpu.VMEM((NB + 1, 1, _CHUNK), jnp.int32),
            pltpu.VMEM((_CHUNK, DC), jnp.float32),
            pltpu.VMEM((_CHUNK, DC), jnp.float32),
            pltpu.VMEM_SHARED((N, DC), jnp.float32),
            pltpu.SemaphoreType.DMA,
            pltpu.SemaphoreType.DMA,
        ],
    )
    def scatter(e_hbm, idx_hbm, z_hbm, out_hbm, idx_v, ebuf0, ebuf1, acc, sem0, sem1):
        c = lax.axis_index("c")
        s = lax.axis_index("s")
        col0 = c * DC
        base = s * NB + jnp.minimum(s, XTRA)
        rb = s * RU

        pltpu.sync_copy(z_hbm, ebuf0)
        for k in range(RU // _CHUNK):
            pltpu.sync_copy(ebuf0, acc.at[pl.ds(rb + k * _CHUNK, _CHUNK), :])
        r = RU % _CHUNK
        if r:
            pltpu.sync_copy(ebuf0.at[pl.ds(0, r), :],
                            acc.at[pl.ds(rb + RU - r, r), :])

        @pl.when(s == 0)
        def _():
            pltpu.sync_copy(ebuf0.at[pl.ds(0, REM), :],
                            acc.at[pl.ds(N - REM, REM), :])

        pltpu.sync_copy(idx_hbm.at[pl.ds(base, NB)], idx_v.at[pl.ds(0, NB)])

        @pl.when(s < XTRA)
        def _():
            pltpu.sync_copy(idx_hbm.at[pl.ds(base + NB, 1)],
                            idx_v.at[pl.ds(NB, 1)])

        pltpu.async_copy(e_hbm.at[base, :, pl.ds(col0, DC)], ebuf0, sem0)

        plsc.subcore_barrier()

        def body(jj, carry):
            q0 = base + jj * 2
            q1 = q0 + 1
            pltpu.make_async_copy(
                e_hbm.at[q0, :, pl.ds(col0, DC)], ebuf0, sem0).wait()
            pltpu.async_copy(e_hbm.at[q1, :, pl.ds(col0, DC)], ebuf1, sem1)
            pltpu.sync_copy(ebuf0, acc.at[idx_v.at[jj * 2, 0]], add=True)
            qn = base + (jj * 2 + 2) % NB
            pltpu.make_async_copy(
                e_hbm.at[q1, :, pl.ds(col0, DC)], ebuf1, sem1).wait()
            pltpu.async_copy(e_hbm.at[qn, :, pl.ds(col0, DC)], ebuf0, sem0)
            pltpu.sync_copy(ebuf1, acc.at[idx_v.at[jj * 2 + 1, 0]], add=True)
            return carry

        lax.fori_loop(0, NB // 2, body, 0)

        pltpu.make_async_copy(
            e_hbm.at[base, :, pl.ds(col0, DC)], ebuf0, sem0).wait()

        @pl.when(s < XTRA)
        def _():
            pltpu.sync_copy(e_hbm.at[base + NB, :, pl.ds(col0, DC)], ebuf0)
            pltpu.sync_copy(ebuf0, acc.at[idx_v.at[NB, 0]], add=True)

        plsc.subcore_barrier()

        pltpu.sync_copy(acc.at[pl.ds(rb, RU), :],
                        out_hbm.at[pl.ds(rb, RU), pl.ds(col0, DC)])

        @pl.when(s == 0)
        def _():
            pltpu.sync_copy(acc.at[pl.ds(N - REM, REM), :],
                            out_hbm.at[pl.ds(N - REM, REM), pl.ds(col0, DC)])

    return scatter(e3, idx3, zeros)


def _mlp(acc, v, W1, b1, W2, b2):
    n, d = v.shape
    br = 1000
    dn = (((1,), (1,)), ((), ()))

    def body(acc_ref, v_ref, w1_ref, b1_ref, w2_ref, b2_ref, out_ref):
        x = acc_ref[...]
        h = lax.dot_general(x, w1_ref[...], dn, preferred_element_type=jnp.float32)
        h = h + b1_ref[...]
        h = jnp.where(h >= 0, h, h * 0.015)
        y = lax.dot_general(h, w2_ref[...], dn, preferred_element_type=jnp.float32)
        out_ref[...] = y + b2_ref[...] + v_ref[...]

    return pl.pallas_call(
        body,
        grid=(n // br,),
        in_specs=[
            pl.BlockSpec((br, d), lambda i: (i, 0)),
            pl.BlockSpec((br, d), lambda i: (i, 0)),
            pl.BlockSpec((d, d), lambda i: (0, 0)),
            pl.BlockSpec((1, d), lambda i: (0, 0)),
            pl.BlockSpec((d, d), lambda i: (0, 0)),
            pl.BlockSpec((1, d), lambda i: (0, 0)),
        ],
        out_specs=pl.BlockSpec((br, d), lambda i: (i, 0)),
        out_shape=jax.ShapeDtypeStruct((n, d), jnp.float32),
    )(acc, v, W1, b1.reshape(1, d), W2, b2.reshape(1, d))


def kernel(v, e, edge_index, W1, b1, W2, b2):
    n, d = v.shape
    ne = e.shape[0]
    e3 = e.reshape(ne // _CHUNK, _CHUNK, d)
    idx3 = edge_index[1].reshape(ne // _CHUNK, 1, _CHUNK)
    zeros = jnp.zeros((_CHUNK, d // 2), jnp.float32)
    acc = _sc_scatter_add(e3, idx3, zeros, n, ne, d)
    return _mlp(acc, v, W1, b1, W2, b2)


# speedup vs baseline: 1.2177x; 1.0130x over previous
import functools

import jax
import jax.numpy as jnp
from jax import lax
from jax.experimental import pallas as pl
from jax.experimental.pallas import tpu as pltpu
from jax.experimental.pallas import tpu_sc as plsc

_CHUNK = 128


@functools.partial(jax.jit, static_argnums=(3, 4, 5))
def _sc_scatter_add(e3, idx3, zeros, N, E, D):
    NC, NS = 2, 16
    DC = D // NC
    NCH = E // _CHUNK
    NB = NCH // NS
    XTRA = NCH - NB * NS
    RU = (N // 16 // NS) * 16
    REM = N - RU * NS
    mesh = plsc.VectorSubcoreMesh(core_axis_name="c", subcore_axis_name="s")

    @functools.partial(
        pl.kernel,
        mesh=mesh,
        out_type=jax.ShapeDtypeStruct((N, D), jnp.float32),
        scratch_types=[
            pltpu.VMEM((NB + 1, 1, _CHUNK), jnp.int32),
            pltpu.VMEM((_CHUNK, DC), jnp.float32),
            pltpu.VMEM((_CHUNK, DC), jnp.float32),
            pltpu.VMEM_SHARED((N, DC), jnp.float32),
            pltpu.SemaphoreType.DMA,
            pltpu.SemaphoreType.DMA,
            pltpu.SemaphoreType.DMA,
            pltpu.SemaphoreType.DMA,
        ],
    )
    def scatter(e_hbm, idx_hbm, z_hbm, out_hbm, idx_v, ebuf0, ebuf1, acc,
                sem0, sem1, sem2, sem3):
        c = lax.axis_index("c")
        s = lax.axis_index("s")
        col0 = c * DC
        base = s * NB + jnp.minimum(s, XTRA)
        rb = s * RU

        pltpu.sync_copy(z_hbm, ebuf0)
        for k in range(RU // _CHUNK):
            pltpu.sync_copy(ebuf0, acc.at[pl.ds(rb + k * _CHUNK, _CHUNK), :])
        r = RU % _CHUNK
        if r:
            pltpu.sync_copy(ebuf0.at[pl.ds(0, r), :],
                            acc.at[pl.ds(rb + RU - r, r), :])

        @pl.when(s == 0)
        def _():
            pltpu.sync_copy(ebuf0.at[pl.ds(0, REM), :],
                            acc.at[pl.ds(N - REM, REM), :])

        pltpu.sync_copy(idx_hbm.at[pl.ds(base, NB)], idx_v.at[pl.ds(0, NB)])

        @pl.when(s < XTRA)
        def _():
            pltpu.sync_copy(idx_hbm.at[pl.ds(base + NB, 1)],
                            idx_v.at[pl.ds(NB, 1)])

        def g_start(q, buf, sa, sb):
            pltpu.async_copy(e_hbm.at[q, pl.ds(0, 64), pl.ds(col0, DC)],
                             buf.at[pl.ds(0, 64), :], sa)
            pltpu.async_copy(e_hbm.at[q, pl.ds(64, 64), pl.ds(col0, DC)],
                             buf.at[pl.ds(64, 64), :], sb)

        def g_wait(q, buf, sa, sb):
            pltpu.make_async_copy(e_hbm.at[q, pl.ds(0, 64), pl.ds(col0, DC)],
                                  buf.at[pl.ds(0, 64), :], sa).wait()
            pltpu.make_async_copy(e_hbm.at[q, pl.ds(64, 64), pl.ds(col0, DC)],
                                  buf.at[pl.ds(64, 64), :], sb).wait()

        g_start(base, ebuf0, sem0, sem2)

        plsc.subcore_barrier()

        def body(jj, carry):
            q0 = base + jj * 2
            q1 = q0 + 1
            g_wait(q0, ebuf0, sem0, sem2)
            g_start(q1, ebuf1, sem1, sem3)
            pltpu.sync_copy(ebuf0, acc.at[idx_v.at[jj * 2, 0]], add=True)
            qn = base + (jj * 2 + 2) % NB
            g_wait(q1, ebuf1, sem1, sem3)
            g_start(qn, ebuf0, sem0, sem2)
            pltpu.sync_copy(ebuf1, acc.at[idx_v.at[jj * 2 + 1, 0]], add=True)
            return carry

        lax.fori_loop(0, NB // 2, body, 0)

        g_wait(base, ebuf0, sem0, sem2)

        @pl.when(s < XTRA)
        def _():
            pltpu.sync_copy(e_hbm.at[base + NB, :, pl.ds(col0, DC)], ebuf0)
            pltpu.sync_copy(ebuf0, acc.at[idx_v.at[NB, 0]], add=True)

        plsc.subcore_barrier()

        pltpu.sync_copy(acc.at[pl.ds(rb, RU), :],
                        out_hbm.at[pl.ds(rb, RU), pl.ds(col0, DC)])

        @pl.when(s == 0)
        def _():
            pltpu.sync_copy(acc.at[pl.ds(N - REM, REM), :],
                            out_hbm.at[pl.ds(N - REM, REM), pl.ds(col0, DC)])

    return scatter(e3, idx3, zeros)


def _mlp(acc, v, W1, b1, W2, b2):
    n, d = v.shape
    br = 1000
    dn = (((1,), (1,)), ((), ()))

    def body(acc_ref, v_ref, w1_ref, b1_ref, w2_ref, b2_ref, out_ref):
        x = acc_ref[...]
        h = lax.dot_general(x, w1_ref[...], dn, preferred_element_type=jnp.float32)
        h = h + b1_ref[...]
        h = jnp.where(h >= 0, h, h * 0.015)
        y = lax.dot_general(h, w2_ref[...], dn, preferred_element_type=jnp.float32)
        out_ref[...] = y + b2_ref[...] + v_ref[...]

    return pl.pallas_call(
        body,
        grid=(n // br,),
        in_specs=[
            pl.BlockSpec((br, d), lambda i: (i, 0)),
            pl.BlockSpec((br, d), lambda i: (i, 0)),
            pl.BlockSpec((d, d), lambda i: (0, 0)),
            pl.BlockSpec((1, d), lambda i: (0, 0)),
            pl.BlockSpec((d, d), lambda i: (0, 0)),
            pl.BlockSpec((1, d), lambda i: (0, 0)),
        ],
        out_specs=pl.BlockSpec((br, d), lambda i: (i, 0)),
        out_shape=jax.ShapeDtypeStruct((n, d), jnp.float32),
    )(acc, v, W1, b1.reshape(1, d), W2, b2.reshape(1, d))


def kernel(v, e, edge_index, W1, b1, W2, b2):
    n, d = v.shape
    ne = e.shape[0]
    e3 = e.reshape(ne // _CHUNK, _CHUNK, d)
    idx3 = edge_index[1].reshape(ne // _CHUNK, 1, _CHUNK)
    zeros = jnp.zeros((_CHUNK, d // 2), jnp.float32)
    acc = _sc_scatter_add(e3, idx3, zeros, n, ne, d)
    return _mlp(acc, v, W1, b1, W2, b2)


# speedup vs baseline: 1.2338x; 1.0132x over previous
import functools

import jax
import jax.numpy as jnp
from jax import lax
from jax.experimental import pallas as pl
from jax.experimental.pallas import tpu as pltpu
from jax.experimental.pallas import tpu_sc as plsc

_CHUNK = 128


@functools.partial(jax.jit, static_argnums=(3, 4, 5))
def _sc_scatter_add(e3, idx3, zeros, N, E, D):
    NC, NS = 2, 16
    DC = D // NC
    NCH = E // _CHUNK
    NB = NCH // NS
    XTRA = NCH - NB * NS
    RU = (N // 16 // NS) * 16
    REM = N - RU * NS
    mesh = plsc.VectorSubcoreMesh(core_axis_name="c", subcore_axis_name="s")

    @functools.partial(
        pl.kernel,
        mesh=mesh,
        out_type=jax.ShapeDtypeStruct((N, D), jnp.float32),
        scratch_types=[
            pltpu.VMEM((NB + 1, 1, _CHUNK), jnp.int32),
            pltpu.VMEM((_CHUNK, DC), jnp.float32),
            pltpu.VMEM((_CHUNK, DC), jnp.float32),
            pltpu.VMEM_SHARED((N, DC), jnp.float32),
            pltpu.SemaphoreType.DMA,
            pltpu.SemaphoreType.DMA,
            pltpu.SemaphoreType.DMA,
            pltpu.SemaphoreType.DMA,
        ],
    )
    def scatter(e_hbm, idx_hbm, z_hbm, out_hbm, idx_v, ebuf0, ebuf1, acc,
                sem0, sem1, sem2, sem3):
        c = lax.axis_index("c")
        s = lax.axis_index("s")
        col0 = c * DC
        base = s * NB + jnp.minimum(s, XTRA)
        rb = s * RU

        pltpu.sync_copy(z_hbm, ebuf0)
        for k in range(RU // _CHUNK):
            pltpu.sync_copy(ebuf0, acc.at[pl.ds(rb + k * _CHUNK, _CHUNK), :])
        r = RU % _CHUNK
        if r:
            pltpu.sync_copy(ebuf0.at[pl.ds(0, r), :],
                            acc.at[pl.ds(rb + RU - r, r), :])

        @pl.when(s == 0)
        def _():
            pltpu.sync_copy(ebuf0.at[pl.ds(0, REM), :],
                            acc.at[pl.ds(N - REM, REM), :])

        pltpu.sync_copy(idx_hbm.at[pl.ds(base, NB)], idx_v.at[pl.ds(0, NB)])

        @pl.when(s < XTRA)
        def _():
            pltpu.sync_copy(idx_hbm.at[pl.ds(base + NB, 1)],
                            idx_v.at[pl.ds(NB, 1)])

        def g_start(q, buf, sa, sb):
            pltpu.async_copy(e_hbm.at[q, pl.ds(0, 64), pl.ds(col0, DC)],
                             buf.at[pl.ds(0, 64), :], sa)
            pltpu.async_copy(e_hbm.at[q, pl.ds(64, 64), pl.ds(col0, DC)],
                             buf.at[pl.ds(64, 64), :], sb)

        def g_wait(q, buf, sa, sb):
            pltpu.make_async_copy(e_hbm.at[q, pl.ds(0, 64), pl.ds(col0, DC)],
                                  buf.at[pl.ds(0, 64), :], sa).wait()
            pltpu.make_async_copy(e_hbm.at[q, pl.ds(64, 64), pl.ds(col0, DC)],
                                  buf.at[pl.ds(64, 64), :], sb).wait()

        g_start(base, ebuf0, sem0, sem2)

        plsc.subcore_barrier()

        def body(jj, carry):
            q0 = base + jj * 2
            q1 = q0 + 1
            g_wait(q0, ebuf0, sem0, sem2)
            g_start(q1, ebuf1, sem1, sem3)
            pltpu.sync_copy(ebuf0, acc.at[idx_v.at[jj * 2, 0]], add=True)
            qn = base + (jj * 2 + 2) % NB
            g_wait(q1, ebuf1, sem1, sem3)
            g_start(qn, ebuf0, sem0, sem2)
            pltpu.sync_copy(ebuf1, acc.at[idx_v.at[jj * 2 + 1, 0]], add=True)
            return carry

        lax.fori_loop(0, NB // 2, body, 0)

        g_wait(base, ebuf0, sem0, sem2)

        @pl.when(s < XTRA)
        def _():
            pltpu.sync_copy(e_hbm.at[base + NB, :, pl.ds(col0, DC)], ebuf0)
            pltpu.sync_copy(ebuf0, acc.at[idx_v.at[NB, 0]], add=True)

        plsc.subcore_barrier()

        pltpu.sync_copy(acc.at[pl.ds(rb, RU), :],
                        out_hbm.at[pl.ds(rb, RU), pl.ds(col0, DC)])

        @pl.when(s == 0)
        def _():
            pltpu.sync_copy(acc.at[pl.ds(N - REM, REM), :],
                            out_hbm.at[pl.ds(N - REM, REM), pl.ds(col0, DC)])

    return scatter(e3, idx3, zeros)


def _mlp(acc, v, W1, b1, W2, b2):
    n, d = v.shape
    br = 2000
    dn = (((1,), (1,)), ((), ()))

    def body(acc_ref, v_ref, w1_ref, b1_ref, w2_ref, b2_ref, out_ref):
        x = acc_ref[...]
        h = lax.dot_general(x, w1_ref[...], dn, preferred_element_type=jnp.float32)
        h = h + b1_ref[...]
        h = jnp.where(h >= 0, h, h * 0.015)
        y = lax.dot_general(h, w2_ref[...], dn, preferred_element_type=jnp.float32)
        out_ref[...] = y + b2_ref[...] + v_ref[...]

    return pl.pallas_call(
        body,
        grid=(n // br,),
        in_specs=[
            pl.BlockSpec((br, d), lambda i: (i, 0)),
            pl.BlockSpec((br, d), lambda i: (i, 0)),
            pl.BlockSpec((d, d), lambda i: (0, 0)),
            pl.BlockSpec((1, d), lambda i: (0, 0)),
            pl.BlockSpec((d, d), lambda i: (0, 0)),
            pl.BlockSpec((1, d), lambda i: (0, 0)),
        ],
        out_specs=pl.BlockSpec((br, d), lambda i: (i, 0)),
        out_shape=jax.ShapeDtypeStruct((n, d), jnp.float32),
    )(acc, v, W1, b1.reshape(1, d), W2, b2.reshape(1, d))


def kernel(v, e, edge_index, W1, b1, W2, b2):
    n, d = v.shape
    ne = e.shape[0]
    e3 = e.reshape(ne // _CHUNK, _CHUNK, d)
    idx3 = edge_index[1].reshape(ne // _CHUNK, 1, _CHUNK)
    zeros = jnp.zeros((_CHUNK, d // 2), jnp.float32)
    acc = _sc_scatter_add(e3, idx3, zeros, n, ne, d)
    return _mlp(acc, v, W1, b1, W2, b2)
